# baseline (speedup 1.0000x reference)
import functools

import jax
import jax.numpy as jnp
from jax import lax
from jax.experimental import pallas as pl
from jax.experimental.pallas import tpu as pltpu
from jax.experimental.pallas import tpu_sc as plsc

NC = 2
NS = 16
NW = NC * NS
CHUNK = 128
RB = 1024


def _sc_degree(dstp, zeros1d, ones_chunk, n_pad):
    k = dstp.shape[1]
    rpt = n_pad // NS

    @functools.partial(
        pl.kernel,
        out_type=jax.ShapeDtypeStruct((NC, n_pad), jnp.float32),
        mesh=plsc.VectorSubcoreMesh(core_axis_name="c", subcore_axis_name="s"),
        scratch_types=[
            pltpu.VMEM((k, CHUNK), jnp.int32),
            pltpu.VMEM((CHUNK,), jnp.float32),
            pltpu.VMEM_SHARED((n_pad,), jnp.float32),
        ],
    )
    def run(dst_hbm, z_hbm, ones_hbm, out_hbm, idx_v, ones_v, acc):
        c = lax.axis_index("c")
        s = lax.axis_index("s")
        w = c * NS + s
        pltpu.sync_copy(z_hbm, acc.at[pl.ds(s * rpt, rpt)])
        pltpu.sync_copy(dst_hbm.at[w], idx_v)
        pltpu.sync_copy(ones_hbm, ones_v)
        plsc.subcore_barrier()

        def body(j, carry):
            pltpu.sync_copy(ones_v, acc.at[idx_v.at[j]], add=True)
            return carry

        lax.fori_loop(0, k, body, 0)
        plsc.subcore_barrier()
        pltpu.sync_copy(acc.at[pl.ds(s * rpt, rpt)],
                        out_hbm.at[c, pl.ds(s * rpt, rpt)])

    return run(dstp, zeros1d, ones_chunk)


def _sc_prop(g, srcp, dstp, zeros_rows):
    n_pad, d = g.shape
    k = srcp.shape[1]
    rpt = n_pad // NS

    @functools.partial(
        pl.kernel,
        out_type=jax.ShapeDtypeStruct((NC, n_pad, d), jnp.float32),
        mesh=plsc.VectorSubcoreMesh(core_axis_name="c", subcore_axis_name="s"),
        scratch_types=[
            pltpu.VMEM((k // 2, CHUNK), jnp.int32),
            pltpu.VMEM((k // 2, CHUNK), jnp.int32),
            pltpu.VMEM((CHUNK, d), jnp.float32),
            pltpu.VMEM((CHUNK, d), jnp.float32),
            pltpu.VMEM_SHARED((n_pad, d), jnp.float32),
            pltpu.SemaphoreType.DMA,
            pltpu.SemaphoreType.DMA,
        ],
    )
    def run(g_hbm, src_hbm, dst_hbm, z_hbm, out_hbm,
            src_v, dst_v, rows0_v, rows1_v, acc, sem_g, sem_s):
        c = lax.axis_index("c")
        s = lax.axis_index("s")
        w = c * NS + s
        kh = k // 2
        pltpu.sync_copy(z_hbm, acc.at[pl.ds(s * rpt, rpt)])

        for h in range(2):
            pltpu.sync_copy(src_hbm.at[w, pl.ds(h * kh, kh)], src_v)
            pltpu.sync_copy(dst_hbm.at[w, pl.ds(h * kh, kh)], dst_v)
            if h == 0:
                plsc.subcore_barrier()

            def body(jj, carry):
                j0 = 2 * jj
                g0 = pltpu.async_copy(g_hbm.at[src_v.at[j0]], rows0_v, sem_g)
                g1 = pltpu.async_copy(g_hbm.at[src_v.at[j0 + 1]], rows1_v, sem_g)
                g0.wait()
                s0 = pltpu.async_copy(rows0_v, acc.at[dst_v.at[j0]], sem_s, add=True)
                g1.wait()
                s1 = pltpu.async_copy(rows1_v, acc.at[dst_v.at[j0 + 1]], sem_s, add=True)
                s0.wait()
                s1.wait()
                return carry

            lax.fori_loop(0, kh // 2, body, 0)
        plsc.subcore_barrier()
        pltpu.sync_copy(acc.at[pl.ds(s * rpt, rpt)],
                        out_hbm.at[c, pl.ds(s * rpt, rpt)])

    return run(g, srcp, dstp, zeros_rows)


def _tc_matmul_scale(xp, w, dinvm):
    n_pad, d = xp.shape
    grid = (n_pad // RB,)

    def body(x_ref, w_ref, di_ref, o_ref):
        xw = jnp.dot(x_ref[...], w_ref[...], preferred_element_type=jnp.float32)
        o_ref[...] = xw * di_ref[...]

    return pl.pallas_call(
        body,
        grid=grid,
        in_specs=[
            pl.BlockSpec((RB, d), lambda i: (i, 0)),
            pl.BlockSpec((d, d), lambda i: (0, 0)),
            pl.BlockSpec((RB, d), lambda i: (i, 0)),
        ],
        out_specs=pl.BlockSpec((RB, d), lambda i: (i, 0)),
        out_shape=jax.ShapeDtypeStruct((n_pad, d), jnp.float32),
    )(xp, w, dinvm)


def _tc_layer(s1, g1, dinvm, b1r):
    n_pad, d = g1.shape
    grid = (n_pad // RB,)

    def body(s_ref, g_ref, di_ref, b_ref, o_ref):
        pre = di_ref[...] * (s_ref[0] + s_ref[1] + g_ref[...]) + b_ref[...]
        o_ref[...] = di_ref[...] * jnp.maximum(pre, 0.0)

    return pl.pallas_call(
        body,
        grid=grid,
        in_specs=[
            pl.BlockSpec((NC, RB, d), lambda i: (0, i, 0)),
            pl.BlockSpec((RB, d), lambda i: (i, 0)),
            pl.BlockSpec((RB, d), lambda i: (i, 0)),
            pl.BlockSpec((1, d), lambda i: (0, 0)),
        ],
        out_specs=pl.BlockSpec((RB, d), lambda i: (i, 0)),
        out_shape=jax.ShapeDtypeStruct((n_pad, d), jnp.float32),
    )(s1, g1, dinvm, b1r)


def _tc_final(s2, g2, dinvm, wcat, bcatr):
    n_pad, d = g2.shape
    grid = (n_pad // RB,)

    def body(s_ref, g_ref, di_ref, w_ref, b_ref, o_ref):
        q = di_ref[...] * (s_ref[0] + s_ref[1] + g_ref[...])
        o_ref[...] = jnp.dot(q, w_ref[...], preferred_element_type=jnp.float32) + b_ref[...]

    return pl.pallas_call(
        body,
        grid=grid,
        in_specs=[
            pl.BlockSpec((NC, RB, d), lambda i: (0, i, 0)),
            pl.BlockSpec((RB, d), lambda i: (i, 0)),
            pl.BlockSpec((RB, d), lambda i: (i, 0)),
            pl.BlockSpec((d, d), lambda i: (0, 0)),
            pl.BlockSpec((1, d), lambda i: (0, 0)),
        ],
        out_specs=pl.BlockSpec((RB, d), lambda i: (i, 0)),
        out_shape=jax.ShapeDtypeStruct((n_pad, d), jnp.float32),
    )(s2, g2, dinvm, wcat, bcatr)


def kernel(x, edge_index, W1, b1, Wmu, bmu, Wls, bls):
    n, din = x.shape
    e = edge_index.shape[1]
    dh = W1.shape[1]
    dout = Wmu.shape[1]

    n_pad = ((n + RB - 1) // RB) * RB
    per_op = NW * CHUNK
    k = (e + per_op - 1) // per_op
    k = ((k + 3) // 4) * 4
    e_pad = NW * k * CHUNK

    src = jnp.concatenate([edge_index[0], jnp.zeros((e_pad - e,), jnp.int32)])
    pad_dst = n + jnp.arange(e_pad - e, dtype=jnp.int32) % (n_pad - n)
    dst = jnp.concatenate([edge_index[1], pad_dst])
    srcp = src.reshape(NW, k, CHUNK)
    dstp = dst.reshape(NW, k, CHUNK)
    xp = jnp.concatenate([x, jnp.zeros((n_pad - n, din), x.dtype)])

    rpt = n_pad // NS
    zeros1d = jnp.zeros((rpt,), jnp.float32)
    zeros_rows = jnp.zeros((rpt, dh), jnp.float32)
    ones_chunk = jnp.ones((CHUNK,), jnp.float32)

    degp = _sc_degree(dstp, zeros1d, ones_chunk, n_pad)
    deg = degp[0] + degp[1] + 1.0
    dinv = lax.rsqrt(deg)
    dinvm = jnp.broadcast_to(dinv[:, None], (n_pad, dh))

    g1 = _tc_matmul_scale(xp, W1, dinvm)
    s1 = _sc_prop(g1, srcp, dstp, zeros_rows)
    g2 = _tc_layer(s1, g1, dinvm, b1.reshape(1, dh))
    s2 = _sc_prop(g2, srcp, dstp, zeros_rows)

    wcat = jnp.concatenate([Wmu, Wls], axis=1)
    bcat = jnp.concatenate([bmu, bls]).reshape(1, 2 * dout)
    out = _tc_final(s2, g2, dinvm, wcat, bcat)
    return (out[:n, :dout], out[:n, dout:])


# speedup vs baseline: 3.1567x; 3.1567x over previous
import functools

import jax
import jax.numpy as jnp
from jax import lax
from jax.experimental import pallas as pl
from jax.experimental.pallas import tpu as pltpu
from jax.experimental.pallas import tpu_sc as plsc

NC = 2
NS = 16
NW = NC * NS
CHUNK = 128
RB = 1024


def _sc_degree(dstp, zeros1d, ones_chunk, n_pad):
    k = dstp.shape[1]
    rpt = n_pad // NS

    @functools.partial(
        pl.kernel,
        out_type=jax.ShapeDtypeStruct((NC, n_pad), jnp.float32),
        mesh=plsc.VectorSubcoreMesh(core_axis_name="c", subcore_axis_name="s"),
        scratch_types=[
            pltpu.VMEM((k, CHUNK), jnp.int32),
            pltpu.VMEM((CHUNK,), jnp.float32),
            pltpu.VMEM_SHARED((n_pad,), jnp.float32),
        ],
    )
    def run(dst_hbm, z_hbm, ones_hbm, out_hbm, idx_v, ones_v, acc):
        c = lax.axis_index("c")
        s = lax.axis_index("s")
        w = c * NS + s
        pltpu.sync_copy(z_hbm, acc.at[pl.ds(s * rpt, rpt)])
        pltpu.sync_copy(dst_hbm.at[w], idx_v)
        pltpu.sync_copy(ones_hbm, ones_v)
        plsc.subcore_barrier()

        def body(j, carry):
            pltpu.sync_copy(ones_v, acc.at[idx_v.at[j]], add=True)
            return carry

        lax.fori_loop(0, k, body, 0)
        plsc.subcore_barrier()
        pltpu.sync_copy(acc.at[pl.ds(s * rpt, rpt)],
                        out_hbm.at[c, pl.ds(s * rpt, rpt)])

    return run(dstp, zeros1d, ones_chunk)


def _sc_prop(g, srcp, dstp, zeros_rows):
    n_pad, d = g.shape
    k = srcp.shape[1]
    rpt = n_pad // NS

    @functools.partial(
        pl.kernel,
        out_type=jax.ShapeDtypeStruct((NC, n_pad, d), jnp.float32),
        mesh=plsc.VectorSubcoreMesh(core_axis_name="c", subcore_axis_name="s"),
        scratch_types=[
            pltpu.VMEM((k // 2, CHUNK), jnp.int32),
            pltpu.VMEM((k // 2, CHUNK), jnp.int32),
            pltpu.VMEM((CHUNK, d), jnp.float32),
            pltpu.VMEM((CHUNK, d), jnp.float32),
            pltpu.VMEM_SHARED((n_pad, d), jnp.float32),
            pltpu.SemaphoreType.DMA,
            pltpu.SemaphoreType.DMA,
        ],
    )
    def run(g_hbm, src_hbm, dst_hbm, z_hbm, out_hbm,
            src_v, dst_v, rows0_v, rows1_v, acc, sem_g, sem_s):
        c = lax.axis_index("c")
        s = lax.axis_index("s")
        w = c * NS + s
        kh = k // 2
        pltpu.sync_copy(z_hbm, acc.at[pl.ds(s * rpt, rpt)])

        for h in range(2):
            pltpu.sync_copy(src_hbm.at[w, pl.ds(h * kh, kh)], src_v)
            pltpu.sync_copy(dst_hbm.at[w, pl.ds(h * kh, kh)], dst_v)
            if h == 0:
                plsc.subcore_barrier()

            def body(jj, carry):
                j0 = 2 * jj
                g0 = pltpu.async_copy(g_hbm.at[src_v.at[j0]], rows0_v, sem_g)
                g1 = pltpu.async_copy(g_hbm.at[src_v.at[j0 + 1]], rows1_v, sem_g)
                g0.wait()
                s0 = pltpu.async_copy(rows0_v, acc.at[dst_v.at[j0]], sem_s, add=True)
                g1.wait()
                s1 = pltpu.async_copy(rows1_v, acc.at[dst_v.at[j0 + 1]], sem_s, add=True)
                s0.wait()
                s1.wait()
                return carry

            lax.fori_loop(0, kh // 2, body, 0)
        plsc.subcore_barrier()
        pltpu.sync_copy(acc.at[pl.ds(s * rpt, rpt)],
                        out_hbm.at[c, pl.ds(s * rpt, rpt)])

    return run(g, srcp, dstp, zeros_rows)


def _tc_matmul_scale(xp, w, dinvm):
    n_pad, d = xp.shape
    grid = (n_pad // RB,)

    def body(x_ref, w_ref, di_ref, o_ref):
        xw = jnp.dot(x_ref[...], w_ref[...], preferred_element_type=jnp.float32)
        o_ref[...] = xw * di_ref[...]

    return pl.pallas_call(
        body,
        grid=grid,
        in_specs=[
            pl.BlockSpec((RB, d), lambda i: (i, 0)),
            pl.BlockSpec((d, d), lambda i: (0, 0)),
            pl.BlockSpec((RB, d), lambda i: (i, 0)),
        ],
        out_specs=pl.BlockSpec((RB, d), lambda i: (i, 0)),
        out_shape=jax.ShapeDtypeStruct((n_pad, d), jnp.float32),
    )(xp, w, dinvm)


def _tc_layer(s1, g1, dinvm, b1r):
    n_pad, d = g1.shape
    grid = (n_pad // RB,)

    def body(s_ref, g_ref, di_ref, b_ref, o_ref):
        pre = di_ref[...] * (s_ref[0] + s_ref[1] + g_ref[...]) + b_ref[...]
        o_ref[...] = di_ref[...] * jnp.maximum(pre, 0.0)

    return pl.pallas_call(
        body,
        grid=grid,
        in_specs=[
            pl.BlockSpec((NC, RB, d), lambda i: (0, i, 0)),
            pl.BlockSpec((RB, d), lambda i: (i, 0)),
            pl.BlockSpec((RB, d), lambda i: (i, 0)),
            pl.BlockSpec((1, d), lambda i: (0, 0)),
        ],
        out_specs=pl.BlockSpec((RB, d), lambda i: (i, 0)),
        out_shape=jax.ShapeDtypeStruct((n_pad, d), jnp.float32),
    )(s1, g1, dinvm, b1r)


def _tc_final(s2, g2, dinvm, wcat, bcatr):
    n_pad, d = g2.shape
    grid = (n_pad // RB,)

    def body(s_ref, g_ref, di_ref, w_ref, b_ref, o_ref):
        q = di_ref[...] * (s_ref[0] + s_ref[1] + g_ref[...])
        o_ref[...] = jnp.dot(q, w_ref[...], preferred_element_type=jnp.float32) + b_ref[...]

    return pl.pallas_call(
        body,
        grid=grid,
        in_specs=[
            pl.BlockSpec((NC, RB, d), lambda i: (0, i, 0)),
            pl.BlockSpec((RB, d), lambda i: (i, 0)),
            pl.BlockSpec((RB, d), lambda i: (i, 0)),
            pl.BlockSpec((d, d), lambda i: (0, 0)),
            pl.BlockSpec((1, d), lambda i: (0, 0)),
        ],
        out_specs=pl.BlockSpec((RB, d), lambda i: (i, 0)),
        out_shape=jax.ShapeDtypeStruct((n_pad, d), jnp.float32),
    )(s2, g2, dinvm, wcat, bcatr)


def kernel(x, edge_index, W1, b1, Wmu, bmu, Wls, bls):
    n, din = x.shape
    e = edge_index.shape[1]
    dh = W1.shape[1]
    dout = Wmu.shape[1]

    n_pad = ((n + RB - 1) // RB) * RB
    per_op = NW * CHUNK
    k = (e + per_op - 1) // per_op
    k = ((k + 3) // 4) * 4
    e_pad = NW * k * CHUNK

    pad_ar = jnp.arange(e_pad - e, dtype=jnp.int32)
    src = jnp.concatenate([edge_index[0], pad_ar % n])
    dst = jnp.concatenate([edge_index[1], n + pad_ar % (n_pad - n)])
    srcp = src.reshape(NW, k, CHUNK)
    dstp = dst.reshape(NW, k, CHUNK)
    xp = jnp.concatenate([x, jnp.zeros((n_pad - n, din), x.dtype)])

    rpt = n_pad // NS
    zeros1d = jnp.zeros((rpt,), jnp.float32)
    zeros_rows = jnp.zeros((rpt, dh), jnp.float32)
    ones_chunk = jnp.ones((CHUNK,), jnp.float32)

    degp = _sc_degree(dstp, zeros1d, ones_chunk, n_pad)
    deg = degp[0] + degp[1] + 1.0
    dinv = lax.rsqrt(deg)
    dinvm = jnp.broadcast_to(dinv[:, None], (n_pad, dh))

    g1 = _tc_matmul_scale(xp, W1, dinvm)
    s1 = _sc_prop(g1, srcp, dstp, zeros_rows)
    g2 = _tc_layer(s1, g1, dinvm, b1.reshape(1, dh))
    s2 = _sc_prop(g2, srcp, dstp, zeros_rows)

    wcat = jnp.concatenate([Wmu, Wls], axis=1)
    bcat = jnp.concatenate([bmu, bls]).reshape(1, 2 * dout)
    out = _tc_final(s2, g2, dinvm, wcat, bcat)
    return (out[:n, :dout], out[:n, dout:])


# speedup vs baseline: 3.2129x; 1.0178x over previous
import functools

import jax
import jax.numpy as jnp
from jax import lax
from jax.experimental import pallas as pl
from jax.experimental.pallas import tpu as pltpu
from jax.experimental.pallas import tpu_sc as plsc

NC = 2
NS = 16
NW = NC * NS
CHUNK = 128
RB = 1024


def _sc_degree(dstp, zeros1d, ones_chunk, n_pad):
    k = dstp.shape[1]
    rpt = n_pad // NS

    @functools.partial(
        pl.kernel,
        out_type=jax.ShapeDtypeStruct((NC, n_pad), jnp.float32),
        mesh=plsc.VectorSubcoreMesh(core_axis_name="c", subcore_axis_name="s"),
        scratch_types=[
            pltpu.VMEM((k, CHUNK), jnp.int32),
            pltpu.VMEM((CHUNK,), jnp.float32),
            pltpu.VMEM_SHARED((n_pad,), jnp.float32),
        ],
    )
    def run(dst_hbm, z_hbm, ones_hbm, out_hbm, idx_v, ones_v, acc):
        c = lax.axis_index("c")
        s = lax.axis_index("s")
        w = c * NS + s
        pltpu.sync_copy(z_hbm, acc.at[pl.ds(s * rpt, rpt)])
        pltpu.sync_copy(dst_hbm.at[w], idx_v)
        pltpu.sync_copy(ones_hbm, ones_v)
        plsc.subcore_barrier()

        def body(j, carry):
            pltpu.sync_copy(ones_v, acc.at[idx_v.at[j]], add=True)
            return carry

        lax.fori_loop(0, k, body, 0)
        plsc.subcore_barrier()
        pltpu.sync_copy(acc.at[pl.ds(s * rpt, rpt)],
                        out_hbm.at[c, pl.ds(s * rpt, rpt)])

    return run(dstp, zeros1d, ones_chunk)


def _sc_prop(g, srcp, dstp, zeros_rows):
    n_pad, d = g.shape
    k = srcp.shape[1]
    rpt = n_pad // NS

    @functools.partial(
        pl.kernel,
        out_type=jax.ShapeDtypeStruct((NC, n_pad, d), jnp.float32),
        mesh=plsc.VectorSubcoreMesh(core_axis_name="c", subcore_axis_name="s"),
        scratch_types=[
            pltpu.VMEM((k // 2, CHUNK), jnp.int32),
            pltpu.VMEM((k // 2, CHUNK), jnp.int32),
            pltpu.VMEM((CHUNK, d), jnp.float32),
            pltpu.VMEM((CHUNK, d), jnp.float32),
            pltpu.VMEM_SHARED((n_pad, d), jnp.float32),
            pltpu.SemaphoreType.DMA,
            pltpu.SemaphoreType.DMA,
        ],
    )
    def run(g_hbm, src_hbm, dst_hbm, z_hbm, out_hbm,
            src_v, dst_v, rows0_v, rows1_v, acc, sem_g, sem_s):
        c = lax.axis_index("c")
        s = lax.axis_index("s")
        w = c * NS + s
        kh = k // 2
        pltpu.sync_copy(z_hbm, acc.at[pl.ds(s * rpt, rpt)])

        for h in range(2):
            pltpu.sync_copy(src_hbm.at[w, pl.ds(h * kh, kh)], src_v)
            pltpu.sync_copy(dst_hbm.at[w, pl.ds(h * kh, kh)], dst_v)
            if h == 0:
                plsc.subcore_barrier()

            def body(jj, carry):
                j0 = 2 * jj
                g0 = pltpu.async_copy(g_hbm.at[src_v.at[j0]], rows0_v, sem_g)
                g1 = pltpu.async_copy(g_hbm.at[src_v.at[j0 + 1]], rows1_v, sem_g)
                g0.wait()
                s0 = pltpu.async_copy(rows0_v, acc.at[dst_v.at[j0]], sem_s, add=True)
                g1.wait()
                s1 = pltpu.async_copy(rows1_v, acc.at[dst_v.at[j0 + 1]], sem_s, add=True)
                s0.wait()
                s1.wait()
                return carry

            lax.fori_loop(0, kh // 2, body, 0)
        plsc.subcore_barrier()
        pltpu.sync_copy(acc.at[pl.ds(s * rpt, rpt)],
                        out_hbm.at[c, pl.ds(s * rpt, rpt)])

    return run(g, srcp, dstp, zeros_rows)


def _dinv_block(deg_blk):
    v8 = lax.rsqrt(deg_blk[0] + deg_blk[1] + 1.0)
    ii = lax.broadcasted_iota(jnp.int32, (128, 128), 0)
    jj = lax.broadcasted_iota(jnp.int32, (128, 128), 1)
    eye = ii == jj
    ones = jnp.ones((128, 128), jnp.float32)
    blocks = []
    for a in range(v8.shape[0]):
        diag = jnp.where(eye, jnp.broadcast_to(v8[a:a + 1, :], (128, 128)), 0.0)
        blocks.append(jnp.dot(diag, ones, preferred_element_type=jnp.float32))
    return jnp.concatenate(blocks, axis=0)


def _tc_matmul_scale(xp, w, degr):
    n_pad, d = xp.shape
    grid = (n_pad // RB,)

    def body(x_ref, w_ref, deg_ref, o_ref):
        xw = jnp.dot(x_ref[...], w_ref[...], preferred_element_type=jnp.float32)
        o_ref[...] = xw * _dinv_block(deg_ref[...])

    return pl.pallas_call(
        body,
        grid=grid,
        in_specs=[
            pl.BlockSpec((RB, d), lambda i: (i, 0)),
            pl.BlockSpec((d, d), lambda i: (0, 0)),
            pl.BlockSpec((NC, RB // 128, 128), lambda i: (0, i, 0)),
        ],
        out_specs=pl.BlockSpec((RB, d), lambda i: (i, 0)),
        out_shape=jax.ShapeDtypeStruct((n_pad, d), jnp.float32),
    )(xp, w, degr)


def _tc_layer(s1, g1, degr, b1r):
    n_pad, d = g1.shape
    grid = (n_pad // RB,)

    def body(s_ref, g_ref, deg_ref, b_ref, o_ref):
        di = _dinv_block(deg_ref[...])
        pre = di * (s_ref[0] + s_ref[1] + g_ref[...]) + b_ref[...]
        o_ref[...] = di * jnp.maximum(pre, 0.0)

    return pl.pallas_call(
        body,
        grid=grid,
        in_specs=[
            pl.BlockSpec((NC, RB, d), lambda i: (0, i, 0)),
            pl.BlockSpec((RB, d), lambda i: (i, 0)),
            pl.BlockSpec((NC, RB // 128, 128), lambda i: (0, i, 0)),
            pl.BlockSpec((1, d), lambda i: (0, 0)),
        ],
        out_specs=pl.BlockSpec((RB, d), lambda i: (i, 0)),
        out_shape=jax.ShapeDtypeStruct((n_pad, d), jnp.float32),
    )(s1, g1, degr, b1r)


def _tc_final(s2, g2, degr, wcat, bcatr):
    n_pad, d = g2.shape
    grid = (n_pad // RB,)

    def body(s_ref, g_ref, deg_ref, w_ref, b_ref, o_ref):
        q = _dinv_block(deg_ref[...]) * (s_ref[0] + s_ref[1] + g_ref[...])
        o_ref[...] = jnp.dot(q, w_ref[...], preferred_element_type=jnp.float32) + b_ref[...]

    return pl.pallas_call(
        body,
        grid=grid,
        in_specs=[
            pl.BlockSpec((NC, RB, d), lambda i: (0, i, 0)),
            pl.BlockSpec((RB, d), lambda i: (i, 0)),
            pl.BlockSpec((NC, RB // 128, 128), lambda i: (0, i, 0)),
            pl.BlockSpec((d, d), lambda i: (0, 0)),
            pl.BlockSpec((1, d), lambda i: (0, 0)),
        ],
        out_specs=pl.BlockSpec((RB, d), lambda i: (i, 0)),
        out_shape=jax.ShapeDtypeStruct((n_pad, d), jnp.float32),
    )(s2, g2, degr, wcat, bcatr)


def kernel(x, edge_index, W1, b1, Wmu, bmu, Wls, bls):
    n, din = x.shape
    e = edge_index.shape[1]
    dh = W1.shape[1]
    dout = Wmu.shape[1]

    n_pad = ((n + RB - 1) // RB) * RB
    per_op = NW * CHUNK
    k = (e + per_op - 1) // per_op
    k = ((k + 3) // 4) * 4
    e_pad = NW * k * CHUNK

    pad_ar = jnp.arange(e_pad - e, dtype=jnp.int32)
    src = jnp.concatenate([edge_index[0], pad_ar % n])
    dst = jnp.concatenate([edge_index[1], n + pad_ar % (n_pad - n)])
    srcp = src.reshape(NW, k, CHUNK)
    dstp = dst.reshape(NW, k, CHUNK)
    xp = jnp.concatenate([x, jnp.zeros((n_pad - n, din), x.dtype)])

    rpt = n_pad // NS
    zeros1d = jnp.zeros((rpt,), jnp.float32)
    zeros_rows = jnp.zeros((rpt, dh), jnp.float32)
    ones_chunk = jnp.ones((CHUNK,), jnp.float32)

    degp = _sc_degree(dstp, zeros1d, ones_chunk, n_pad)
    degr = degp.reshape(NC, n_pad // 128, 128)

    g1 = _tc_matmul_scale(xp, W1, degr)
    s1 = _sc_prop(g1, srcp, dstp, zeros_rows)
    g2 = _tc_layer(s1, g1, degr, b1.reshape(1, dh))
    s2 = _sc_prop(g2, srcp, dstp, zeros_rows)

    wcat = jnp.concatenate([Wmu, Wls], axis=1)
    bcat = jnp.concatenate([bmu, bls]).reshape(1, 2 * dout)
    out = _tc_final(s2, g2, degr, wcat, bcat)
    return (out[:n, :dout], out[:n, dout:])
